# baseline (speedup 1.0000x reference)
import functools

import numpy as np
import jax
import jax.numpy as jnp
from jax import lax
from jax.experimental import pallas as pl
from jax.experimental.pallas import tpu as pltpu
from jax.experimental.pallas import tpu_sc as plsc
from jax.extend.random import threefry2x32_p

_B = 16384
_P = 200
_NF = 32
_SKIP = 10
_NC = 2
_NS = 16
_NW = _NC * _NS
_L = 16
_CHUNK = _B // _NW
_HALF = _CHUNK // 2
_G = _HALF // _L
_PSPLIT = 100
_BPS = _B // 256

_TINY = np.float32(np.finfo(np.float32).tiny)


def _sample_chunk(logits, p0, p1):
    pc = p1 - p0
    b = lax.broadcasted_iota(jnp.uint32, (_B, pc, _NF), 0)
    p = lax.broadcasted_iota(jnp.uint32, (_B, pc, _NF), 1) + np.uint32(p0)
    k = lax.broadcasted_iota(jnp.uint32, (_B, pc, _NF), 2)
    cnt = b * np.uint32(_P * _NF) + p * np.uint32(_NF) + k
    hi, lo = threefry2x32_p.bind(jnp.uint32(0), jnp.uint32(123),
                                 jnp.zeros_like(cnt).ravel(), cnt.ravel())
    bits = (hi ^ lo).reshape(_B, pc, _NF)
    fb = (bits >> np.uint32(9)) | np.uint32(0x3F800000)
    floats = lax.bitcast_convert_type(fb, jnp.float32) - np.float32(1.0)
    u = lax.max(_TINY, floats * (np.float32(1.0) - _TINY) + _TINY)
    g = -jnp.log(-jnp.log(u))
    return jnp.argmax(g + logits[:, None, :], axis=-1).astype(jnp.int32)


def _ifs_body(n_steps, skip, idx_hbm, pt_hbm, tab_hbm, out_hbm, st_hbm,
              idx_v, tab_v, x_v, y_v, stage_v, sem):
    wid = lax.axis_index("s") * _NC + lax.axis_index("c")
    base = wid * _CHUNK

    pltpu.sync_copy(tab_hbm, tab_v)
    lanes = lax.iota(jnp.int32, _L)

    for h in range(2):
        hb = base + h * _HALF
        pltpu.sync_copy(idx_hbm.at[pl.ds(hb, _HALF), :], idx_v)
        pltpu.sync_copy(pt_hbm.at[0, pl.ds(hb, _HALF)], x_v)
        pltpu.sync_copy(pt_hbm.at[1, pl.ds(hb, _HALF)], y_v)

        def step(p, carry):
            pcol = jnp.full((_L,), p, jnp.int32)

            for g in range(_G):
                rows = g * _L + lanes
                fidx = plsc.load_gather(idx_v, [rows, pcol])
                w00 = plsc.load_gather(tab_v, [fidx])
                w01 = plsc.load_gather(tab_v, [fidx + _NF])
                w10 = plsc.load_gather(tab_v, [fidx + 2 * _NF])
                w11 = plsc.load_gather(tab_v, [fidx + 3 * _NF])
                b0 = plsc.load_gather(tab_v, [fidx + 4 * _NF])
                b1 = plsc.load_gather(tab_v, [fidx + 5 * _NF])
                op = plsc.load_gather(tab_v, [fidx + 6 * _NF])
                x = x_v[pl.ds(g * _L, _L)]
                y = y_v[pl.ds(g * _L, _L)]
                nx = w00 * x + w01 * y + b0
                ny = w10 * x + w11 * y + b1
                x_v[pl.ds(g * _L, _L)] = nx
                y_v[pl.ds(g * _L, _L)] = ny
                sub = 4 * (g // 8)
                col = (g % 8) * _L
                stage_v[sub + 0, pl.ds(col, _L)] = nx
                stage_v[sub + 1, pl.ds(col, _L)] = ny
                stage_v[sub + 2, pl.ds(col, _L)] = op

            @pl.when(p >= skip)
            def _():
                blk = (p - skip) * _BPS + hb // 256
                pltpu.async_copy(stage_v, out_hbm.at[blk], sem).wait()

            return carry

        lax.fori_loop(0, n_steps, step, 0)
        pltpu.sync_copy(x_v, st_hbm.at[0, pl.ds(hb, _HALF)])
        pltpu.sync_copy(y_v, st_hbm.at[1, pl.ds(hb, _HALF)])


def _make_run(n_steps, skip):
    mesh = plsc.VectorSubcoreMesh(core_axis_name="c", subcore_axis_name="s")
    return pl.kernel(
        functools.partial(_ifs_body, n_steps, skip),
        out_type=(
            jax.ShapeDtypeStruct(((n_steps - skip) * _BPS, 8, 128),
                                 jnp.float32),
            jax.ShapeDtypeStruct((2, _B), jnp.float32),
        ),
        mesh=mesh,
        compiler_params=pltpu.CompilerParams(needs_layout_passes=False,
                                             use_tc_tiling_on_sc=True),
        scratch_types=[
            pltpu.VMEM((_HALF, _PSPLIT), jnp.int32),
            pltpu.VMEM((224,), jnp.float32),
            pltpu.VMEM((_HALF,), jnp.float32),
            pltpu.VMEM((_HALF,), jnp.float32),
            pltpu.VMEM((8, 128), jnp.float32),
            pltpu.SemaphoreType.DMA,
        ],
    )


def kernel(point, optimized_weights, optimized_biases, optimized_function_ops,
           code):
    logits = jnp.log(code + 1e-8)
    idx1 = _sample_chunk(logits, 0, _PSPLIT)
    idx2 = _sample_chunk(logits, _PSPLIT, _P)

    tab = jnp.concatenate([
        optimized_weights[:, 0, 0], optimized_weights[:, 0, 1],
        optimized_weights[:, 1, 0], optimized_weights[:, 1, 1],
        optimized_biases[:, 0, 0], optimized_biases[:, 1, 0],
        optimized_function_ops,
    ])
    pt_t = point[:, :, 0].T

    out1, st = _make_run(_PSPLIT, _SKIP)(idx1, pt_t, tab)
    out2, _ = _make_run(_P - _PSPLIT, 0)(idx2, st, tab)

    out = jnp.concatenate([out1, out2], axis=0)
    nrows = (_P - _SKIP) * _B
    return (out.reshape(nrows // 128, 4, 128)
               .transpose(0, 2, 1)[:, :, :3]
               .reshape(nrows, 3))


# speedup vs baseline: 2.8446x; 2.8446x over previous
import functools

import numpy as np
import jax
import jax.numpy as jnp
from jax import lax
from jax.experimental import pallas as pl
from jax.experimental.pallas import tpu as pltpu
from jax.experimental.pallas import tpu_sc as plsc
from jax.extend.random import threefry2x32_p

_B = 16384
_P = 200
_NF = 32
_SKIP = 10
_NC = 2
_NS = 16
_NW = _NC * _NS
_L = 16
_CHUNK = _B // _NW
_HALF = _CHUNK // 2
_G = _HALF // _L
_PSPLIT = 100
_BPS = _B // 256

_TINY = np.float32(np.finfo(np.float32).tiny)


def _sample_chunk(logits, p0, p1):
    pc = p1 - p0
    b = lax.broadcasted_iota(jnp.uint32, (_B, pc, _NF), 0)
    p = lax.broadcasted_iota(jnp.uint32, (_B, pc, _NF), 1) + np.uint32(p0)
    k = lax.broadcasted_iota(jnp.uint32, (_B, pc, _NF), 2)
    cnt = b * np.uint32(_P * _NF) + p * np.uint32(_NF) + k
    hi, lo = threefry2x32_p.bind(jnp.uint32(0), jnp.uint32(123),
                                 jnp.zeros_like(cnt), cnt)
    bits = hi ^ lo
    fb = (bits >> np.uint32(9)) | np.uint32(0x3F800000)
    floats = lax.bitcast_convert_type(fb, jnp.float32) - np.float32(1.0)
    u = lax.max(_TINY, floats * (np.float32(1.0) - _TINY) + _TINY)
    g = -jnp.log(-jnp.log(u))
    return jnp.argmax(g + logits[:, None, :], axis=-1).astype(jnp.int32)


def _ifs_body(n_steps, skip, idx_hbm, pt_hbm, tab_hbm, out_hbm, st_hbm,
              idx_v, tab_v, x_v, y_v, stage_v, sem):
    wid = lax.axis_index("s") * _NC + lax.axis_index("c")
    base = wid * _CHUNK

    pltpu.sync_copy(tab_hbm, tab_v)
    lanes = lax.iota(jnp.int32, _L)

    for h in range(2):
        hb = base + h * _HALF
        pltpu.sync_copy(idx_hbm.at[pl.ds(hb, _HALF), :], idx_v)
        pltpu.sync_copy(pt_hbm.at[0, pl.ds(hb, _HALF)], x_v)
        pltpu.sync_copy(pt_hbm.at[1, pl.ds(hb, _HALF)], y_v)

        def step(p, carry):
            pcol = jnp.full((_L,), p, jnp.int32)

            for g in range(_G):
                rows = g * _L + lanes
                fidx = plsc.load_gather(idx_v, [rows, pcol])
                w00 = plsc.load_gather(tab_v, [fidx])
                w01 = plsc.load_gather(tab_v, [fidx + _NF])
                w10 = plsc.load_gather(tab_v, [fidx + 2 * _NF])
                w11 = plsc.load_gather(tab_v, [fidx + 3 * _NF])
                b0 = plsc.load_gather(tab_v, [fidx + 4 * _NF])
                b1 = plsc.load_gather(tab_v, [fidx + 5 * _NF])
                op = plsc.load_gather(tab_v, [fidx + 6 * _NF])
                x = x_v[pl.ds(g * _L, _L)]
                y = y_v[pl.ds(g * _L, _L)]
                nx = w00 * x + w01 * y + b0
                ny = w10 * x + w11 * y + b1
                x_v[pl.ds(g * _L, _L)] = nx
                y_v[pl.ds(g * _L, _L)] = ny
                sub = 4 * (g // 8)
                col = (g % 8) * _L
                stage_v[sub + 0, pl.ds(col, _L)] = nx
                stage_v[sub + 1, pl.ds(col, _L)] = ny
                stage_v[sub + 2, pl.ds(col, _L)] = op

            @pl.when(p >= skip)
            def _():
                blk = (p - skip) * _BPS + hb // 256
                pltpu.async_copy(stage_v, out_hbm.at[blk], sem).wait()

            return carry

        lax.fori_loop(0, n_steps, step, 0)
        pltpu.sync_copy(x_v, st_hbm.at[0, pl.ds(hb, _HALF)])
        pltpu.sync_copy(y_v, st_hbm.at[1, pl.ds(hb, _HALF)])


def _make_run(n_steps, skip):
    mesh = plsc.VectorSubcoreMesh(core_axis_name="c", subcore_axis_name="s")
    return pl.kernel(
        functools.partial(_ifs_body, n_steps, skip),
        out_type=(
            jax.ShapeDtypeStruct(((n_steps - skip) * _BPS, 8, 128),
                                 jnp.float32),
            jax.ShapeDtypeStruct((2, _B), jnp.float32),
        ),
        mesh=mesh,
        compiler_params=pltpu.CompilerParams(needs_layout_passes=False,
                                             use_tc_tiling_on_sc=True),
        scratch_types=[
            pltpu.VMEM((_HALF, _PSPLIT), jnp.int32),
            pltpu.VMEM((224,), jnp.float32),
            pltpu.VMEM((_HALF,), jnp.float32),
            pltpu.VMEM((_HALF,), jnp.float32),
            pltpu.VMEM((8, 128), jnp.float32),
            pltpu.SemaphoreType.DMA,
        ],
    )


def kernel(point, optimized_weights, optimized_biases, optimized_function_ops,
           code):
    logits = jnp.log(code + 1e-8)
    idx1 = _sample_chunk(logits, 0, _PSPLIT)
    idx2 = _sample_chunk(logits, _PSPLIT, _P)

    tab = jnp.concatenate([
        optimized_weights[:, 0, 0], optimized_weights[:, 0, 1],
        optimized_weights[:, 1, 0], optimized_weights[:, 1, 1],
        optimized_biases[:, 0, 0], optimized_biases[:, 1, 0],
        optimized_function_ops,
    ])
    pt_t = point[:, :, 0].T

    out1, st = _make_run(_PSPLIT, _SKIP)(idx1, pt_t, tab)
    out2, _ = _make_run(_P - _PSPLIT, 0)(idx2, st, tab)

    out = jnp.concatenate([out1, out2], axis=0)
    nrows = (_P - _SKIP) * _B
    return (out.reshape(nrows // 128, 4, 128)
               .transpose(0, 2, 1)[:, :, :3]
               .reshape(nrows, 3))


# speedup vs baseline: 2.9246x; 1.0281x over previous
import functools

import numpy as np
import jax
import jax.numpy as jnp
from jax import lax
from jax.experimental import pallas as pl
from jax.experimental.pallas import tpu as pltpu
from jax.experimental.pallas import tpu_sc as plsc
from jax.extend.random import threefry2x32_p

_B = 16384
_P = 200
_NF = 32
_SKIP = 10
_NC = 2
_NS = 16
_NW = _NC * _NS
_L = 16
_CHUNK = _B // _NW
_HALF = _CHUNK // 2
_G = _HALF // _L
_PSPLIT = 160
_BPS = _B // 256

_TINY = np.float32(np.finfo(np.float32).tiny)


def _sample_chunk(logits, p0, p1):
    pc = p1 - p0
    b = lax.broadcasted_iota(jnp.uint32, (_B, pc, _NF), 0)
    p = lax.broadcasted_iota(jnp.uint32, (_B, pc, _NF), 1) + np.uint32(p0)
    k = lax.broadcasted_iota(jnp.uint32, (_B, pc, _NF), 2)
    cnt = b * np.uint32(_P * _NF) + p * np.uint32(_NF) + k
    hi, lo = threefry2x32_p.bind(jnp.uint32(0), jnp.uint32(123),
                                 jnp.zeros_like(cnt), cnt)
    bits = hi ^ lo
    fb = (bits >> np.uint32(9)) | np.uint32(0x3F800000)
    floats = lax.bitcast_convert_type(fb, jnp.float32) - np.float32(1.0)
    u = lax.max(_TINY, floats * (np.float32(1.0) - _TINY) + _TINY)
    g = -jnp.log(-jnp.log(u))
    return jnp.argmax(g + logits[:, None, :], axis=-1).astype(jnp.int32)


def _ifs_body(n_steps, skip, idx_hbm, pt_hbm, tab_hbm, out_hbm, st_hbm,
              idx_v, tab_v, x_v, y_v, stage_v, sem):
    wid = lax.axis_index("s") * _NC + lax.axis_index("c")
    base = wid * _CHUNK

    pltpu.sync_copy(tab_hbm, tab_v)
    lanes = lax.iota(jnp.int32, _L)

    for h in range(2):
        hb = base + h * _HALF
        pltpu.sync_copy(idx_hbm.at[pl.ds(hb, _HALF), :], idx_v)
        pltpu.sync_copy(pt_hbm.at[0, pl.ds(hb, _HALF)], x_v)
        pltpu.sync_copy(pt_hbm.at[1, pl.ds(hb, _HALF)], y_v)

        def step(p, carry):
            pcol = jnp.full((_L,), p, jnp.int32)

            for g in range(_G):
                rows = g * _L + lanes
                fidx = plsc.load_gather(idx_v, [rows, pcol])
                w00 = plsc.load_gather(tab_v, [fidx])
                w01 = plsc.load_gather(tab_v, [fidx + _NF])
                w10 = plsc.load_gather(tab_v, [fidx + 2 * _NF])
                w11 = plsc.load_gather(tab_v, [fidx + 3 * _NF])
                b0 = plsc.load_gather(tab_v, [fidx + 4 * _NF])
                b1 = plsc.load_gather(tab_v, [fidx + 5 * _NF])
                op = plsc.load_gather(tab_v, [fidx + 6 * _NF])
                x = x_v[pl.ds(g * _L, _L)]
                y = y_v[pl.ds(g * _L, _L)]
                nx = w00 * x + w01 * y + b0
                ny = w10 * x + w11 * y + b1
                x_v[pl.ds(g * _L, _L)] = nx
                y_v[pl.ds(g * _L, _L)] = ny
                sub = 4 * (g // 8)
                col = (g % 8) * _L
                stage_v[sub + 0, pl.ds(col, _L)] = nx
                stage_v[sub + 1, pl.ds(col, _L)] = ny
                stage_v[sub + 2, pl.ds(col, _L)] = op

            @pl.when(p >= skip)
            def _():
                blk = (p - skip) * _BPS + hb // 256
                pltpu.async_copy(stage_v, out_hbm.at[blk], sem).wait()

            return carry

        lax.fori_loop(0, n_steps, step, 0)
        pltpu.sync_copy(x_v, st_hbm.at[0, pl.ds(hb, _HALF)])
        pltpu.sync_copy(y_v, st_hbm.at[1, pl.ds(hb, _HALF)])


def _make_run(n_steps, skip):
    mesh = plsc.VectorSubcoreMesh(core_axis_name="c", subcore_axis_name="s")
    return pl.kernel(
        functools.partial(_ifs_body, n_steps, skip),
        out_type=(
            jax.ShapeDtypeStruct(((n_steps - skip) * _BPS, 8, 128),
                                 jnp.float32),
            jax.ShapeDtypeStruct((2, _B), jnp.float32),
        ),
        mesh=mesh,
        compiler_params=pltpu.CompilerParams(needs_layout_passes=False,
                                             use_tc_tiling_on_sc=True),
        scratch_types=[
            pltpu.VMEM((_HALF, n_steps), jnp.int32),
            pltpu.VMEM((224,), jnp.float32),
            pltpu.VMEM((_HALF,), jnp.float32),
            pltpu.VMEM((_HALF,), jnp.float32),
            pltpu.VMEM((8, 128), jnp.float32),
            pltpu.SemaphoreType.DMA,
        ],
    )


def kernel(point, optimized_weights, optimized_biases, optimized_function_ops,
           code):
    logits = jnp.log(code + 1e-8)
    idx1 = _sample_chunk(logits, 0, _PSPLIT)
    idx2 = _sample_chunk(logits, _PSPLIT, _P)

    tab = jnp.concatenate([
        optimized_weights[:, 0, 0], optimized_weights[:, 0, 1],
        optimized_weights[:, 1, 0], optimized_weights[:, 1, 1],
        optimized_biases[:, 0, 0], optimized_biases[:, 1, 0],
        optimized_function_ops,
    ])
    pt_t = point[:, :, 0].T

    out1, st = _make_run(_PSPLIT, _SKIP)(idx1, pt_t, tab)
    out2, _ = _make_run(_P - _PSPLIT, 0)(idx2, st, tab)

    out = jnp.concatenate([out1, out2], axis=0)
    nrows = (_P - _SKIP) * _B
    return (out.reshape(nrows // 128, 4, 128)
               .transpose(0, 2, 1)[:, :, :3]
               .reshape(nrows, 3))
